# baseline (speedup 1.0000x reference)
import functools
import jax
import jax.numpy as jnp
from jax import lax
from jax.experimental import pallas as pl
from jax.experimental.pallas import tpu as pltpu
from jax.experimental.pallas import tpu_sc as plsc


def _make_sc_kernel(B, maxlen, D, W):
    info = plsc.get_sparse_core_info()
    NC, NS, L = info.num_cores, info.num_subcores, info.num_lanes
    NW = NC * NS
    C = 128
    n_phases = maxlen // C
    n_chunks = (B * maxlen) // (NW * C)
    n_pairs = n_chunks // 2
    n_tg = C // L
    mesh = plsc.VectorSubcoreMesh(core_axis_name="c", subcore_axis_name="s")

    @functools.partial(
        pl.kernel,
        mesh=mesh,
        compiler_params=pltpu.CompilerParams(
            use_tc_tiling_on_sc=True, needs_layout_passes=False
        ),
        out_type=jax.ShapeDtypeStruct((B, D, maxlen), jnp.float32),
        scratch_types=[
            pltpu.VMEM((192 * C,), jnp.int32),
            pltpu.VMEM((C,), jnp.int32),
            pltpu.VMEM((C,), jnp.int32),
            pltpu.VMEM((C,), jnp.int32),
            pltpu.VMEM((n_phases, D, C), jnp.float32),
            pltpu.VMEM((C, W), jnp.float32),
            pltpu.VMEM((C, W), jnp.float32),
            pltpu.VMEM((D, C), jnp.float32),
            pltpu.VMEM((D, C), jnp.float32),
            pltpu.SemaphoreType.DMA,
            pltpu.SemaphoreType.DMA,
            pltpu.SemaphoreType.DMA,
            pltpu.SemaphoreType.DMA,
        ],
    )
    def k(x_hbm, tok_hbm, pos_hbm, out_hbm,
          idx_all, idxg0, idxg1, cadj, pos_v, rows0, rows1, plane0, plane1,
          semg0, semg1, sems0, sems1):
        wid = lax.axis_index("s") * NC + lax.axis_index("c")
        first = wid * n_chunks
        pltpu.sync_copy(pos_hbm, pos_v)

        row_iota = jax.lax.iota(jnp.int32, L)

        def prep_gather_list(c_local, idxg):
            for tg in range(n_tg):
                v = idx_all[pl.ds(c_local * C + tg * L, L)]
                idxg[pl.ds(tg * L, L)] = jax.lax.shift_right_logical(v, 1)

        def compute_store(rows, plane, sem, chunk, c_local):
            batch = chunk // n_phases
            phase = lax.rem(chunk, n_phases)
            dst = out_hbm.at[batch, :, pl.ds(phase * C, C)]

            @pl.when(chunk >= first + 2)
            def _():
                pltpu.make_async_copy(plane, dst, sem).wait()

            for tg in range(n_tg):
                v = idx_all[pl.ds(c_local * C + tg * L, L)]
                cadj[pl.ds(tg * L, L)] = jax.lax.shift_left(v & 1, 6)

            @plsc.parallel_loop(0, D, 1, unroll=2)
            def init_loop(f):
                for tg in range(n_tg):
                    plane[f, pl.ds(tg * L, L)] = pos_v[
                        phase, f, pl.ds(tg * L, L)
                    ]

            for tg in range(n_tg):
                cbase = cadj[pl.ds(tg * L, L)]
                ridx = row_iota + tg * L

                @plsc.parallel_loop(0, D, 1, unroll=2)
                def tr_loop(f):
                    vals = plsc.load_gather(rows, [ridx, cbase + f])
                    plsc.addupdate(plane.at[f, pl.ds(tg * L, L)], vals)

            pltpu.async_copy(plane, dst, sem)

        pltpu.sync_copy(x_hbm.at[pl.ds(first * C, n_chunks * C)], idx_all)

        prep_gather_list(0, idxg0)
        pltpu.async_copy(tok_hbm.at[idxg0], rows0, semg0)

        def pair_body(j, carry):
            ca = first + 2 * j
            prep_gather_list(2 * j + 1, idxg1)
            pltpu.async_copy(tok_hbm.at[idxg1], rows1, semg1)
            pltpu.make_async_copy(tok_hbm.at[idxg0], rows0, semg0).wait()
            compute_store(rows0, plane0, sems0, ca, 2 * j)
            @pl.when(j + 1 < n_pairs)
            def _():
                prep_gather_list(2 * j + 2, idxg0)
                pltpu.async_copy(tok_hbm.at[idxg0], rows0, semg0)
            pltpu.make_async_copy(tok_hbm.at[idxg1], rows1, semg1).wait()
            compute_store(rows1, plane1, sems1, ca + 1, 2 * j + 1)
            return carry

        lax.fori_loop(0, n_pairs, pair_body, 0)

        last = first + n_chunks - 1
        for plane, sem, chunk in ((plane0, sems0, last - 1), (plane1, sems1, last)):
            batch = chunk // n_phases
            phase = chunk % n_phases
            pltpu.make_async_copy(
                plane, out_hbm.at[batch, :, pl.ds(phase * C, C)], sem
            ).wait()

    return k


def kernel(x, token_table, pos_table):
    B, maxlen = x.shape
    V, D = token_table.shape
    W = 2 * D
    x_flat = x.reshape(-1).astype(jnp.int32)
    tok_p = token_table.reshape(V // 2, W)
    pos_p = pos_table.T.reshape(D, maxlen // 128, 128).swapaxes(0, 1)
    k = _make_sc_kernel(B, maxlen, D, W)
    out_t = k(x_flat, tok_p, pos_p)
    return out_t.transpose(0, 2, 1)


# speedup vs baseline: 1.0686x; 1.0686x over previous
import functools
import jax
import jax.numpy as jnp
from jax import lax
from jax.experimental import pallas as pl
from jax.experimental.pallas import tpu as pltpu
from jax.experimental.pallas import tpu_sc as plsc


def _make_sc_kernel(B, maxlen, D, W):
    info = plsc.get_sparse_core_info()
    NC, NS, L = info.num_cores, info.num_subcores, info.num_lanes
    NW = NC * NS
    C = 128
    n_phases = maxlen // C
    n_chunks = (B * maxlen) // (NW * C)
    n_pairs = n_chunks // 2
    n_tg = C // L
    mesh = plsc.VectorSubcoreMesh(core_axis_name="c", subcore_axis_name="s")

    @functools.partial(
        pl.kernel,
        mesh=mesh,
        compiler_params=pltpu.CompilerParams(
            use_tc_tiling_on_sc=True, needs_layout_passes=False
        ),
        out_type=jax.ShapeDtypeStruct((B, D, maxlen), jnp.float32),
        scratch_types=[
            pltpu.VMEM((192 * C,), jnp.int32),
            pltpu.VMEM((n_phases, D, C), jnp.float32),
            pltpu.VMEM((C, W), jnp.float32),
            pltpu.VMEM((C, W), jnp.float32),
            pltpu.VMEM((D, C), jnp.float32),
            pltpu.VMEM((D, C), jnp.float32),
            pltpu.SemaphoreType.DMA,
            pltpu.SemaphoreType.DMA,
            pltpu.SemaphoreType.DMA,
            pltpu.SemaphoreType.DMA,
        ],
    )
    def k(x_hbm, tok_hbm, pos_hbm, out_hbm,
          idx_all, pos_v, rows0, rows1, plane0, plane1,
          semg0, semg1, sems0, sems1):
        wid = lax.axis_index("s") * NC + lax.axis_index("c")
        first = wid * n_chunks
        pltpu.sync_copy(pos_hbm, pos_v)

        row_iota = jax.lax.iota(jnp.int32, L)

        def compute_store(rows, plane, sem, chunk):
            batch = chunk // n_phases
            phase = lax.rem(chunk, n_phases)
            dst = out_hbm.at[batch, :, pl.ds(phase * C, C)]

            @pl.when(chunk >= first + 2)
            def _():
                pltpu.make_async_copy(plane, dst, sem).wait()

            @plsc.parallel_loop(0, D, 1, unroll=2)
            def init_loop(f):
                for tg in range(n_tg):
                    plane[f, pl.ds(tg * L, L)] = pos_v[
                        phase, f, pl.ds(tg * L, L)
                    ]

            @plsc.parallel_loop(0, D, 1, unroll=2)
            def tr_loop(f):
                cvec = jnp.full((L,), f, jnp.int32)
                for tg in range(n_tg):
                    vals = plsc.load_gather(rows, [row_iota + tg * L, cvec])
                    plsc.addupdate(plane.at[f, pl.ds(tg * L, L)], vals)

            pltpu.async_copy(plane, dst, sem)

        pltpu.sync_copy(x_hbm.at[pl.ds(first * C, n_chunks * C)], idx_all)

        def gidx(c_local):
            return idx_all.at[pl.ds(c_local * C, C)]

        pltpu.async_copy(tok_hbm.at[gidx(0)], rows0, semg0)

        def pair_body(j, carry):
            ca = first + 2 * j
            pltpu.async_copy(tok_hbm.at[gidx(2 * j + 1)], rows1, semg1)
            pltpu.make_async_copy(tok_hbm.at[gidx(2 * j)], rows0, semg0).wait()
            compute_store(rows0, plane0, sems0, ca)
            @pl.when(j + 1 < n_pairs)
            def _():
                pltpu.async_copy(tok_hbm.at[gidx(2 * j + 2)], rows0, semg0)
            pltpu.make_async_copy(tok_hbm.at[gidx(2 * j + 1)], rows1, semg1).wait()
            compute_store(rows1, plane1, sems1, ca + 1)
            return carry

        lax.fori_loop(0, n_pairs, pair_body, 0)

        last = first + n_chunks - 1
        for plane, sem, chunk in ((plane0, sems0, last - 1), (plane1, sems1, last)):
            batch = chunk // n_phases
            phase = chunk % n_phases
            pltpu.make_async_copy(
                plane, out_hbm.at[batch, :, pl.ds(phase * C, C)], sem
            ).wait()

    return k


def kernel(x, token_table, pos_table):
    B, maxlen = x.shape
    V, D = token_table.shape
    W = 2 * D
    x_flat = x.reshape(-1).astype(jnp.int32)
    tok_p = jnp.pad(token_table, ((0, 0), (0, W - D)))
    pos_p = pos_table.T.reshape(D, maxlen // 128, 128).swapaxes(0, 1)
    k = _make_sc_kernel(B, maxlen, D, W)
    out_t = k(x_flat, tok_p, pos_p)
    return out_t.transpose(0, 2, 1)


# speedup vs baseline: 1.0715x; 1.0027x over previous
import functools
import jax
import jax.numpy as jnp
from jax import lax
from jax.experimental import pallas as pl
from jax.experimental.pallas import tpu as pltpu
from jax.experimental.pallas import tpu_sc as plsc


def _make_sc_kernel(B, maxlen, D, W):
    info = plsc.get_sparse_core_info()
    NC, NS, L = info.num_cores, info.num_subcores, info.num_lanes
    NW = NC * NS
    C = 128
    n_phases = maxlen // C
    n_chunks = (B * maxlen) // (NW * C)
    n_pairs = n_chunks // 2
    n_tg = C // L
    mesh = plsc.VectorSubcoreMesh(core_axis_name="c", subcore_axis_name="s")

    @functools.partial(
        pl.kernel,
        mesh=mesh,
        compiler_params=pltpu.CompilerParams(
            use_tc_tiling_on_sc=True, needs_layout_passes=False
        ),
        out_type=jax.ShapeDtypeStruct((B, D, maxlen), jnp.float32),
        scratch_types=[
            pltpu.VMEM((192 * C,), jnp.int32),
            pltpu.VMEM((n_phases, D, C), jnp.float32),
            pltpu.VMEM((C, W), jnp.float32),
            pltpu.VMEM((C, W), jnp.float32),
            pltpu.VMEM((D, C), jnp.float32),
            pltpu.VMEM((D, C), jnp.float32),
            pltpu.SemaphoreType.DMA,
            pltpu.SemaphoreType.DMA,
            pltpu.SemaphoreType.DMA,
            pltpu.SemaphoreType.DMA,
        ],
    )
    def k(x_hbm, tok_hbm, pos_hbm, out_hbm,
          idx_all, pos_v, rows0, rows1, plane0, plane1,
          semg0, semg1, sems0, sems1):
        wid = lax.axis_index("s") * NC + lax.axis_index("c")
        first = wid * n_chunks
        pltpu.sync_copy(pos_hbm, pos_v)

        row_iota = jax.lax.iota(jnp.int32, L)

        def compute_store(rows, plane, sem, chunk):
            batch = chunk // n_phases
            phase = lax.rem(chunk, n_phases)
            dst = out_hbm.at[batch, :, pl.ds(phase * C, C)]

            @pl.when(chunk >= first + 2)
            def _():
                pltpu.make_async_copy(plane, dst, sem).wait()

            @plsc.parallel_loop(0, D, 1, unroll=2)
            def init_loop(f):
                for tg in range(n_tg):
                    plane[f, pl.ds(tg * L, L)] = pos_v[
                        phase, f, pl.ds(tg * L, L)
                    ]

            @plsc.parallel_loop(0, D, 1, unroll=4)
            def tr_loop(f):
                cvec = jnp.full((L,), f, jnp.int32)
                for tg in range(n_tg):
                    vals = plsc.load_gather(rows, [row_iota + tg * L, cvec])
                    plsc.addupdate(plane.at[f, pl.ds(tg * L, L)], vals)

            pltpu.async_copy(plane, dst, sem)

        pltpu.sync_copy(x_hbm.at[pl.ds(first * C, n_chunks * C)], idx_all)

        def gidx(c_local):
            return idx_all.at[pl.ds(c_local * C, C)]

        pltpu.async_copy(tok_hbm.at[gidx(0)], rows0, semg0)

        def pair_body(j, carry):
            ca = first + 2 * j
            pltpu.async_copy(tok_hbm.at[gidx(2 * j + 1)], rows1, semg1)
            pltpu.make_async_copy(tok_hbm.at[gidx(2 * j)], rows0, semg0).wait()
            compute_store(rows0, plane0, sems0, ca)
            @pl.when(j + 1 < n_pairs)
            def _():
                pltpu.async_copy(tok_hbm.at[gidx(2 * j + 2)], rows0, semg0)
            pltpu.make_async_copy(tok_hbm.at[gidx(2 * j + 1)], rows1, semg1).wait()
            compute_store(rows1, plane1, sems1, ca + 1)
            return carry

        lax.fori_loop(0, n_pairs, pair_body, 0)

        last = first + n_chunks - 1
        for plane, sem, chunk in ((plane0, sems0, last - 1), (plane1, sems1, last)):
            batch = chunk // n_phases
            phase = chunk % n_phases
            pltpu.make_async_copy(
                plane, out_hbm.at[batch, :, pl.ds(phase * C, C)], sem
            ).wait()

    return k


def kernel(x, token_table, pos_table):
    B, maxlen = x.shape
    V, D = token_table.shape
    W = 2 * D
    x_flat = x.reshape(-1).astype(jnp.int32)
    tok_p = jnp.pad(token_table, ((0, 0), (0, W - D)))
    pos_p = pos_table.T.reshape(D, maxlen // 128, 128).swapaxes(0, 1)
    k = _make_sc_kernel(B, maxlen, D, W)
    out_t = k(x_flat, tok_p, pos_p)
    return out_t.transpose(0, 2, 1)
